# baseline (speedup 1.0000x reference)
import math

import jax
import jax.numpy as jnp
from jax import lax
from jax.experimental import pallas as pl
from jax.experimental.pallas import tpu as pltpu
from jax.experimental.pallas import tpu_sc as plsc

D = 64
L = 16
NC, NS = 2, 16
NW = NC * NS
R, S = 16384, 50
RPW = R // NW
P = 4
CH = P * S
NCHUNK = RPW // P
SCALE = math.sqrt(D)


def _emb_body(table_hbm, idx_hbm, off_hbm, out_hbm,
              idx0, idx1, off0, off1, rows0, rows1, blk, sem0, sem1):
    wid = lax.axis_index("s") * NC + lax.axis_index("c")
    rbase = wid * RPW
    fbase = rbase * S
    idx = (idx0, idx1)
    off = (off0, off1)
    rows = (rows0, rows1)
    sems = (sem0, sem1)

    for b in range(2):
        pltpu.sync_copy(idx_hbm.at[pl.ds(fbase + b * CH, CH)], idx[b])
        pltpu.sync_copy(off_hbm.at[pl.ds(fbase + b * CH, CH)], off[b].at[pl.ds(0, CH)])
        pltpu.async_copy(table_hbm.at[idx[b]], rows[b], sems[b])

    @pl.loop(0, NCHUNK, step=2)
    def _chunks(g):
        for b in range(2):
            gb = g + b
            pltpu.make_async_copy(
                table_hbm.at[idx[b]], rows[b], sems[b]).wait()

            def _do_row(p, s, r, h):
                for j in range(D // L):
                    blk[p, s, pl.ds(j * L, L)] = (
                        rows[b][r, pl.ds(h + j * L, L)] * SCALE)

            for p in range(P):
                @plsc.parallel_loop(0, (S // L), 1)
                def _scale_grp(g):
                    s0 = g * L
                    hv = off[b][pl.ds(p * S + s0, L)]
                    for k in range(L):
                        _do_row(p, s0 + k, p * S + s0 + k, hv[k])
                t0 = (S // L) * L
                hv = off[b][pl.ds(p * S + t0, L)]
                for k in range(S - t0):
                    _do_row(p, t0 + k, p * S + t0 + k, hv[k])

            pltpu.sync_copy(blk, out_hbm.at[pl.ds(rbase + gb * P, P)])

            @pl.when(gb + 2 < NCHUNK)
            def _fire():
                nxt = fbase + (gb + 2) * CH
                pltpu.sync_copy(idx_hbm.at[pl.ds(nxt, CH)], idx[b])
                pltpu.sync_copy(off_hbm.at[pl.ds(nxt, CH)], off[b].at[pl.ds(0, CH)])
                pltpu.async_copy(table_hbm.at[idx[b]], rows[b], sems[b])


def kernel(tokens, table):
    tok_flat = tokens.reshape(-1)
    idx2 = jax.lax.shift_right_logical(tok_flat, 1)
    hoff = jax.lax.shift_left(jax.lax.bitwise_and(tok_flat, 1), 6)
    table2 = table.reshape(500000, 128)
    mesh = plsc.VectorSubcoreMesh(core_axis_name="c", subcore_axis_name="s")
    k = pl.kernel(
        _emb_body,
        out_type=jax.ShapeDtypeStruct((R, S, D), jnp.float32),
        mesh=mesh,
        scratch_types=[
            pltpu.VMEM((CH,), jnp.int32),
            pltpu.VMEM((CH,), jnp.int32),
            pltpu.VMEM((CH + L,), jnp.int32),
            pltpu.VMEM((CH + L,), jnp.int32),
            pltpu.VMEM((CH, 128), jnp.float32),
            pltpu.VMEM((CH, 128), jnp.float32),
            pltpu.VMEM((P, S, D), jnp.float32),
            pltpu.SemaphoreType.DMA,
            pltpu.SemaphoreType.DMA,
        ],
    )
    return k(table2, idx2, hoff)


# speedup vs baseline: 1.7074x; 1.7074x over previous
import math

import jax
import jax.numpy as jnp
from jax import lax
from jax.experimental import pallas as pl
from jax.experimental.pallas import tpu as pltpu
from jax.experimental.pallas import tpu_sc as plsc

D = 64
L = 16
NC, NS = 2, 16
NW = NC * NS
R, S = 16384, 50
RPW = R // NW
P = 4
CH = P * S
NCHUNK = RPW // P
SCALE = math.sqrt(D)


def _emb_body(table_hbm, idx_hbm, out_hbm,
              idx0, idx1, rows0, rows1, blk, sem0, sem1):
    wid = lax.axis_index("s") * NC + lax.axis_index("c")
    rbase = wid * RPW
    fbase = rbase * S
    idx = (idx0, idx1)
    rows = (rows0, rows1)
    sems = (sem0, sem1)

    for b in range(2):
        pltpu.sync_copy(idx_hbm.at[pl.ds(fbase + b * CH, CH)], idx[b])
        pltpu.async_copy(table_hbm.at[idx[b]], rows[b], sems[b])

    @pl.loop(0, NCHUNK, step=2)
    def _chunks(g):
        for b in range(2):
            gb = g + b
            pltpu.make_async_copy(
                table_hbm.at[idx[b]], rows[b], sems[b]).wait()

            for p in range(P):
                @plsc.parallel_loop(0, S, 1, unroll=4)
                def _scale_tok(s):
                    r = p * S + s
                    for j in range(D // L):
                        blk[p, s, pl.ds(j * L, L)] = (
                            rows[b][r, pl.ds(j * L, L)] * SCALE)

            pltpu.sync_copy(blk, out_hbm.at[pl.ds(rbase + gb * P, P)])

            @pl.when(gb + 2 < NCHUNK)
            def _fire():
                nxt = fbase + (gb + 2) * CH
                pltpu.sync_copy(idx_hbm.at[pl.ds(nxt, CH)], idx[b])
                pltpu.async_copy(table_hbm.at[idx[b]], rows[b], sems[b])


def kernel(tokens, table):
    tok_flat = tokens.reshape(-1)
    wide = jnp.concatenate([table, table], axis=1)
    mesh = plsc.VectorSubcoreMesh(core_axis_name="c", subcore_axis_name="s")
    k = pl.kernel(
        _emb_body,
        out_type=jax.ShapeDtypeStruct((R, S, D), jnp.float32),
        mesh=mesh,
        scratch_types=[
            pltpu.VMEM((CH,), jnp.int32),
            pltpu.VMEM((CH,), jnp.int32),
            pltpu.VMEM((CH, 128), jnp.float32),
            pltpu.VMEM((CH, 128), jnp.float32),
            pltpu.VMEM((P, S, D), jnp.float32),
            pltpu.SemaphoreType.DMA,
            pltpu.SemaphoreType.DMA,
        ],
    )
    return k(wide, tok_flat)
